# baseline (speedup 1.0000x reference)
import jax
import jax.numpy as jnp
from jax.experimental import pallas as pl
from jax.experimental.pallas import tpu as pltpu

S = 7
C = 20
NBOX = 2
CELL = C + NBOX * 5
BATCH = 8
MID = 2048
IN_F = 1024 * S * S
OUT_F = S * S * CELL
K_BLK = 1024
K_TILES = IN_F // K_BLK
J = 2
MID_BLK = MID // J


def _head_kernel(x_ref, w1_ref, b1_ref, w2_ref, b2_ref, out_ref,
                 acc_ref, oacc_ref):
    j = pl.program_id(0)
    k = pl.program_id(1)

    @pl.when(k == 0)
    def _init():
        acc_ref[...] = jnp.broadcast_to(b1_ref[...], acc_ref.shape)

    acc_ref[...] += jnp.dot(
        x_ref[...], w1_ref[...], preferred_element_type=jnp.float32
    )

    @pl.when(k == K_TILES - 1)
    def _reduce():
        h = acc_ref[...]
        h = jnp.where(h > 0, h, 0.1 * h)
        part = jnp.dot(h, w2_ref[...], preferred_element_type=jnp.float32)

        @pl.when(j == 0)
        def _first():
            oacc_ref[...] = part

        @pl.when(j > 0)
        def _rest():
            oacc_ref[...] += part

        @pl.when(j == J - 1)
        def _finish():
            o = oacc_ref[...] + b2_ref[...]
            col = jax.lax.broadcasted_iota(jnp.int32, o.shape, 1)
            r = col % CELL
            m = (r < C) & ((r % 5 == 1) | (r % 5 == 2))
            out_ref[...] = jnp.where(m, jax.nn.sigmoid(o), o)


def kernel(x, W1, b1, W2, b2):
    x2 = x.reshape(BATCH, IN_F)
    out = pl.pallas_call(
        _head_kernel,
        grid=(J, K_TILES),
        in_specs=[
            pl.BlockSpec((BATCH, K_BLK), lambda j, k: (0, k)),
            pl.BlockSpec((K_BLK, MID_BLK), lambda j, k: (k, j)),
            pl.BlockSpec((1, MID_BLK), lambda j, k: (0, j)),
            pl.BlockSpec((MID_BLK, OUT_F), lambda j, k: (j, 0)),
            pl.BlockSpec((1, OUT_F), lambda j, k: (0, 0)),
        ],
        out_specs=pl.BlockSpec((BATCH, OUT_F), lambda j, k: (0, 0)),
        out_shape=jax.ShapeDtypeStruct((BATCH, OUT_F), jnp.float32),
        scratch_shapes=[
            pltpu.VMEM((BATCH, MID_BLK), jnp.float32),
            pltpu.VMEM((BATCH, OUT_F), jnp.float32),
        ],
        compiler_params=pltpu.CompilerParams(
            dimension_semantics=("arbitrary", "arbitrary"),
        ),
    )(x2, W1, b1[None, :], W2, b2[None, :])
    return out.reshape(-1, S, S, CELL)


# speedup vs baseline: 1.0783x; 1.0783x over previous
import jax
import jax.numpy as jnp
from jax.experimental import pallas as pl
from jax.experimental.pallas import tpu as pltpu

S = 7
C = 20
NBOX = 2
CELL = C + NBOX * 5
BATCH = 8
MID = 2048
IN_F = 1024 * S * S
OUT_F = S * S * CELL
NQ = 2
K_BLK = 896
STEPS = IN_F // (NQ * K_BLK)


def _head_kernel(*refs):
    x_refs = refs[:NQ]
    w_refs = refs[NQ:2 * NQ]
    b1_ref, w2_ref, b2_ref, out_ref, acc_ref = refs[2 * NQ:]
    k = pl.program_id(0)

    @pl.when(k == 0)
    def _init():
        acc_ref[...] = jnp.broadcast_to(b1_ref[...], acc_ref.shape)

    acc = acc_ref[...]
    for q in range(NQ):
        acc += jnp.dot(x_refs[q][...], w_refs[q][...],
                       preferred_element_type=jnp.float32)
    acc_ref[...] = acc

    @pl.when(k == STEPS - 1)
    def _finish():
        h = acc_ref[...]
        h = jnp.where(h > 0, h, 0.1 * h)
        o = jnp.dot(h, w2_ref[...], preferred_element_type=jnp.float32)
        o = o + b2_ref[...]
        col = jax.lax.broadcasted_iota(jnp.int32, o.shape, 1)
        r = col % CELL
        m = (r < C) & ((r % 5 == 1) | (r % 5 == 2))
        out_ref[...] = jnp.where(m, jax.nn.sigmoid(o), o)


def _x_map(q):
    return lambda k: (0, k + q * STEPS)


def _w_map(q):
    return lambda k: (k + q * STEPS, 0)


def kernel(x, W1, b1, W2, b2):
    x2 = x.reshape(BATCH, IN_F)
    in_specs = (
        [pl.BlockSpec((BATCH, K_BLK), _x_map(q)) for q in range(NQ)]
        + [pl.BlockSpec((K_BLK, MID), _w_map(q)) for q in range(NQ)]
        + [
            pl.BlockSpec((1, MID), lambda k: (0, 0)),
            pl.BlockSpec((MID, OUT_F), lambda k: (0, 0)),
            pl.BlockSpec((1, OUT_F), lambda k: (0, 0)),
        ]
    )
    out = pl.pallas_call(
        _head_kernel,
        grid=(STEPS,),
        in_specs=in_specs,
        out_specs=pl.BlockSpec((BATCH, OUT_F), lambda k: (0, 0)),
        out_shape=jax.ShapeDtypeStruct((BATCH, OUT_F), jnp.float32),
        scratch_shapes=[pltpu.VMEM((BATCH, MID), jnp.float32)],
        compiler_params=pltpu.CompilerParams(
            dimension_semantics=("arbitrary",),
        ),
    )(*([x2] * NQ + [W1] * NQ + [b1[None, :], W2, b2[None, :]]))
    return out.reshape(-1, S, S, CELL)


# speedup vs baseline: 1.0792x; 1.0009x over previous
import jax
import jax.numpy as jnp
from jax.experimental import pallas as pl
from jax.experimental.pallas import tpu as pltpu

S = 7
C = 20
NBOX = 2
CELL = C + NBOX * 5
BATCH = 8
MID = 2048
IN_F = 1024 * S * S
OUT_F = S * S * CELL
NQ = 2
K_BLK = 896
STEPS = IN_F // (NQ * K_BLK)


def _head_kernel(*refs):
    x_refs = refs[:NQ]
    w_refs = refs[NQ:2 * NQ]
    b1_ref, w2_ref, b2_ref, out_ref, acc_ref = refs[2 * NQ:]
    k = pl.program_id(0)

    @pl.when(k == 0)
    def _init():
        acc_ref[...] = jnp.broadcast_to(b1_ref[...], acc_ref.shape)

    acc = acc_ref[...]
    for q in range(NQ):
        acc += jnp.dot(x_refs[q][...].astype(jnp.bfloat16),
                       w_refs[q][...].astype(jnp.bfloat16),
                       preferred_element_type=jnp.float32)
    acc_ref[...] = acc

    @pl.when(k == STEPS - 1)
    def _finish():
        h = acc_ref[...]
        h = jnp.where(h > 0, h, 0.1 * h)
        o = jnp.dot(h, w2_ref[...], preferred_element_type=jnp.float32)
        o = o + b2_ref[...]
        col = jax.lax.broadcasted_iota(jnp.int32, o.shape, 1)
        r = col % CELL
        m = (r < C) & ((r % 5 == 1) | (r % 5 == 2))
        out_ref[...] = jnp.where(m, jax.nn.sigmoid(o), o)


def _x_map(q):
    return lambda k: (0, k + q * STEPS)


def _w_map(q):
    return lambda k: (k + q * STEPS, 0)


def kernel(x, W1, b1, W2, b2):
    x2 = x.reshape(BATCH, IN_F)
    in_specs = (
        [pl.BlockSpec((BATCH, K_BLK), _x_map(q)) for q in range(NQ)]
        + [pl.BlockSpec((K_BLK, MID), _w_map(q)) for q in range(NQ)]
        + [
            pl.BlockSpec((1, MID), lambda k: (0, 0)),
            pl.BlockSpec((MID, OUT_F), lambda k: (0, 0)),
            pl.BlockSpec((1, OUT_F), lambda k: (0, 0)),
        ]
    )
    out = pl.pallas_call(
        _head_kernel,
        grid=(STEPS,),
        in_specs=in_specs,
        out_specs=pl.BlockSpec((BATCH, OUT_F), lambda k: (0, 0)),
        out_shape=jax.ShapeDtypeStruct((BATCH, OUT_F), jnp.float32),
        scratch_shapes=[pltpu.VMEM((BATCH, MID), jnp.float32)],
        compiler_params=pltpu.CompilerParams(
            dimension_semantics=("arbitrary",),
        ),
    )(*([x2] * NQ + [W1] * NQ + [b1[None, :], W2, b2[None, :]]))
    return out.reshape(-1, S, S, CELL)
